# baseline (speedup 1.0000x reference)
import functools

import jax
import jax.numpy as jnp
from jax import lax
from jax.experimental import pallas as pl
from jax.experimental.pallas import tpu as pltpu
from jax.experimental.pallas import tpu_sc as plsc

N = 10000
E = 160000
D = 256
D2 = 2 * D

NW = 32
EPW0 = 4096
EPW1 = 6144
E_PAD = 16 * (EPW0 + EPW1)
CH = 32
LANES = 16
_MASK_HI = jnp.int32(-65536)


def _pack_pair(lo_f32, hi_f32):
    lo = lax.bitcast_convert_type(
        lo_f32.astype(jnp.bfloat16), jnp.uint16).astype(jnp.uint32)
    hi = lax.bitcast_convert_type(
        hi_f32.astype(jnp.bfloat16), jnp.uint16).astype(jnp.uint32)
    return lax.bitcast_convert_type(lo | (hi << jnp.uint32(16)), jnp.int32)


def _uv_body(zs, zo, zi, wit, wot, bi, bo, u, v):
    a_in = jnp.dot(zs[...], wit[...], preferred_element_type=jnp.float32) + bi[...]
    a_out = jnp.dot(zs[...], wot[...], preferred_element_type=jnp.float32) + bo[...]
    u[...] = _pack_pair(zo[...], a_in)
    v[...] = _pack_pair(0.5 * a_out, 0.5 * zi[...])


def _build_uv(z_self, z_out, z_in, W_in_T, W_out_T, b_in, b_out):
    R = 1000
    grid = (N // R,)
    return pl.pallas_call(
        _uv_body,
        grid=grid,
        in_specs=[
            pl.BlockSpec((R, D), lambda i: (i, 0)),
            pl.BlockSpec((R, D), lambda i: (i, 0)),
            pl.BlockSpec((R, D), lambda i: (i, 0)),
            pl.BlockSpec((D, D), lambda i: (0, 0)),
            pl.BlockSpec((D, D), lambda i: (0, 0)),
            pl.BlockSpec((1, D), lambda i: (0, 0)),
            pl.BlockSpec((1, D), lambda i: (0, 0)),
        ],
        out_specs=[
            pl.BlockSpec((R, D), lambda i: (i, 0)),
            pl.BlockSpec((R, D), lambda i: (i, 0)),
        ],
        out_shape=[
            jax.ShapeDtypeStruct((N, D), jnp.int32),
            jax.ShapeDtypeStruct((N, D), jnp.int32),
        ],
    )(z_self, z_out, z_in, W_in_T, W_out_T, b_in, b_out)


def _sc_body(u_hbm, v_hbm, src_hbm, dst_hbm, out_hbm,
             idx_s_all, idx_d_all, rows_u0, rows_v0, rows_u1, rows_v1,
             accs, out_v, sem_u0, sem_v0, sem_u1, sem_v1):
    ci = lax.axis_index("c")
    si = lax.axis_index("s")
    lane_iota = lax.iota(jnp.int32, LANES)

    def issue(c, ru, rv, su, sv):
        pltpu.async_copy(u_hbm.at[idx_s_all.at[pl.ds(c * CH, CH)]], ru, su)
        pltpu.async_copy(v_hbm.at[idx_d_all.at[pl.ds(c * CH, CH)]], rv, sv)

    def drain(ru, rv, su, sv):
        pltpu.make_async_copy(u_hbm.at[pl.ds(0, CH)], ru, su).wait()
        pltpu.make_async_copy(v_hbm.at[pl.ds(0, CH)], rv, sv).wait()

    def compute(c, ru, rv):
        for g in range(CH // LANES):
            def edge_body(e, carry2):
                r = g * LANES + e
                lanes_acc = [None, None, None, None]
                for j in range(D2 // (2 * LANES)):
                    uw = ru[r, pl.ds(j * LANES, LANES)]
                    vw = rv[r, pl.ds(j * LANES, LANES)]
                    ue = plsc.bitcast(lax.shift_left(uw, 16), jnp.float32)
                    uo = plsc.bitcast(jnp.bitwise_and(uw, _MASK_HI), jnp.float32)
                    ve = plsc.bitcast(lax.shift_left(vw, 16), jnp.float32)
                    vo = plsc.bitcast(jnp.bitwise_and(vw, _MASK_HI), jnp.float32)
                    k = 2 * (j % 2)
                    pe = ue * ve
                    po = uo * vo
                    lanes_acc[k] = pe if lanes_acc[k] is None else lanes_acc[k] + pe
                    lanes_acc[k + 1] = po if lanes_acc[k + 1] is None else lanes_acc[k + 1] + po
                acc = ((lanes_acc[0] + lanes_acc[1])
                       + (lanes_acc[2] + lanes_acc[3]))
                accs[e, :] = acc
                return carry2

            lax.fori_loop(0, LANES, edge_body, 0)
            tot = plsc.load_gather(
                accs, [lane_iota, jnp.zeros((LANES,), jnp.int32)])
            for l in range(1, LANES):
                tot = tot + plsc.load_gather(
                    accs, [lane_iota, jnp.full((LANES,), l, jnp.int32)])
            sig = 1.0 / (1.0 + jnp.exp(-tot))
            out_v[pl.ds(c * CH + g * LANES, LANES)] = sig

    def work(base, epw):
        nchunk = epw // CH
        pltpu.sync_copy(src_hbm.at[pl.ds(base, epw)], idx_s_all.at[pl.ds(0, epw)])
        pltpu.sync_copy(dst_hbm.at[pl.ds(base, epw)], idx_d_all.at[pl.ds(0, epw)])
        issue(0, rows_u0, rows_v0, sem_u0, sem_v0)

        def pair_body(t, carry):
            c0 = 2 * t
            c1 = 2 * t + 1
            issue(c1, rows_u1, rows_v1, sem_u1, sem_v1)
            drain(rows_u0, rows_v0, sem_u0, sem_v0)
            compute(c0, rows_u0, rows_v0)

            @pl.when(c1 + 1 < nchunk)
            def _():
                issue(c1 + 1, rows_u0, rows_v0, sem_u0, sem_v0)

            drain(rows_u1, rows_v1, sem_u1, sem_v1)
            compute(c1, rows_u1, rows_v1)
            return carry

        lax.fori_loop(0, nchunk // 2, pair_body, 0)
        pltpu.sync_copy(out_v.at[pl.ds(0, epw)], out_hbm.at[pl.ds(base, epw)])

    @pl.when(ci == 0)
    def _():
        work(si * EPW0, EPW0)

    @pl.when(ci == 1)
    def _():
        work(16 * EPW0 + si * EPW1, EPW1)


_sc_edge_scores = functools.partial(
    pl.kernel,
    mesh=plsc.VectorSubcoreMesh(core_axis_name="c", subcore_axis_name="s"),
    compiler_params=pltpu.CompilerParams(needs_layout_passes=False),
    out_type=jax.ShapeDtypeStruct((E_PAD,), jnp.float32),
    scratch_types=[
        pltpu.VMEM((EPW1,), jnp.int32),
        pltpu.VMEM((EPW1,), jnp.int32),
        pltpu.VMEM((CH, D2 // 2), jnp.int32),
        pltpu.VMEM((CH, D2 // 2), jnp.int32),
        pltpu.VMEM((CH, D2 // 2), jnp.int32),
        pltpu.VMEM((CH, D2 // 2), jnp.int32),
        pltpu.VMEM((LANES, LANES), jnp.float32),
        pltpu.VMEM((EPW1,), jnp.float32),
        pltpu.SemaphoreType.DMA,
        pltpu.SemaphoreType.DMA,
        pltpu.SemaphoreType.DMA,
        pltpu.SemaphoreType.DMA,
    ],
)(_sc_body)


def kernel(z_in, z_out, z_self, edge_index, W_in, b_in, W_out, b_out):
    U, V = _build_uv(z_self, z_out, z_in,
                     W_in.T, W_out.T,
                     b_in.reshape(1, D), b_out.reshape(1, D))
    ei = edge_index.astype(jnp.int32)
    pad = jnp.zeros((E_PAD - E,), jnp.int32)
    src = jnp.concatenate([ei[0], pad])
    dst = jnp.concatenate([ei[1], pad])
    out = _sc_edge_scores(U, V, src, dst)
    return out[:E]


# speedup vs baseline: 1.2358x; 1.2358x over previous
import functools

import jax
import jax.numpy as jnp
from jax import lax
from jax.experimental import pallas as pl
from jax.experimental.pallas import tpu as pltpu
from jax.experimental.pallas import tpu_sc as plsc

N = 10000
E = 160000
D = 256
D2 = 2 * D

NW = 32
EPW0 = 6144
EPW1 = 4096
E_PAD = 16 * (EPW0 + EPW1)
CH = 32
EPWMAX = max(EPW0, EPW1)
LANES = 16
_MASK_HI = jnp.int32(-65536)


def _pack_pair(lo_f32, hi_f32):
    lo = lax.bitcast_convert_type(
        lo_f32.astype(jnp.bfloat16), jnp.uint16).astype(jnp.uint32)
    hi = lax.bitcast_convert_type(
        hi_f32.astype(jnp.bfloat16), jnp.uint16).astype(jnp.uint32)
    return lax.bitcast_convert_type(lo | (hi << jnp.uint32(16)), jnp.int32)


def _uv_body(zs, zo, zi, wit, wot, bi, bo, u, v):
    a_in = jnp.dot(zs[...], wit[...], preferred_element_type=jnp.float32) + bi[...]
    a_out = jnp.dot(zs[...], wot[...], preferred_element_type=jnp.float32) + bo[...]
    u[...] = _pack_pair(zo[...], a_in)
    v[...] = _pack_pair(0.5 * a_out, 0.5 * zi[...])


def _build_uv(z_self, z_out, z_in, W_in_T, W_out_T, b_in, b_out):
    R = 1000
    grid = (N // R,)
    return pl.pallas_call(
        _uv_body,
        grid=grid,
        in_specs=[
            pl.BlockSpec((R, D), lambda i: (i, 0)),
            pl.BlockSpec((R, D), lambda i: (i, 0)),
            pl.BlockSpec((R, D), lambda i: (i, 0)),
            pl.BlockSpec((D, D), lambda i: (0, 0)),
            pl.BlockSpec((D, D), lambda i: (0, 0)),
            pl.BlockSpec((1, D), lambda i: (0, 0)),
            pl.BlockSpec((1, D), lambda i: (0, 0)),
        ],
        out_specs=[
            pl.BlockSpec((R, D), lambda i: (i, 0)),
            pl.BlockSpec((R, D), lambda i: (i, 0)),
        ],
        out_shape=[
            jax.ShapeDtypeStruct((N, D), jnp.int32),
            jax.ShapeDtypeStruct((N, D), jnp.int32),
        ],
    )(z_self, z_out, z_in, W_in_T, W_out_T, b_in, b_out)


def _sc_body(u_hbm, v_hbm, src_hbm, dst_hbm, out_hbm,
             idx_s_all, idx_d_all, rows_u0, rows_v0, rows_u1, rows_v1,
             accs, out_v, sem_u0, sem_v0, sem_u1, sem_v1):
    ci = lax.axis_index("c")
    si = lax.axis_index("s")
    lane_iota = lax.iota(jnp.int32, LANES)

    def issue(c, ru, rv, su, sv):
        pltpu.async_copy(u_hbm.at[idx_s_all.at[pl.ds(c * CH, CH)]], ru, su)
        pltpu.async_copy(v_hbm.at[idx_d_all.at[pl.ds(c * CH, CH)]], rv, sv)

    def drain(ru, rv, su, sv):
        pltpu.make_async_copy(u_hbm.at[pl.ds(0, CH)], ru, su).wait()
        pltpu.make_async_copy(v_hbm.at[pl.ds(0, CH)], rv, sv).wait()

    def compute(c, ru, rv):
        for g in range(CH // LANES):
            def edge_body(e, carry2):
                r = g * LANES + e
                lanes_acc = [None, None, None, None]
                for j in range(D2 // (2 * LANES)):
                    uw = ru[r, pl.ds(j * LANES, LANES)]
                    vw = rv[r, pl.ds(j * LANES, LANES)]
                    ue = plsc.bitcast(lax.shift_left(uw, 16), jnp.float32)
                    uo = plsc.bitcast(jnp.bitwise_and(uw, _MASK_HI), jnp.float32)
                    ve = plsc.bitcast(lax.shift_left(vw, 16), jnp.float32)
                    vo = plsc.bitcast(jnp.bitwise_and(vw, _MASK_HI), jnp.float32)
                    k = 2 * (j % 2)
                    pe = ue * ve
                    po = uo * vo
                    lanes_acc[k] = pe if lanes_acc[k] is None else lanes_acc[k] + pe
                    lanes_acc[k + 1] = po if lanes_acc[k + 1] is None else lanes_acc[k + 1] + po
                acc = ((lanes_acc[0] + lanes_acc[1])
                       + (lanes_acc[2] + lanes_acc[3]))
                accs[e, :] = acc
                return carry2

            lax.fori_loop(0, LANES, edge_body, 0)
            tot = plsc.load_gather(
                accs, [lane_iota, jnp.zeros((LANES,), jnp.int32)])
            for l in range(1, LANES):
                tot = tot + plsc.load_gather(
                    accs, [lane_iota, jnp.full((LANES,), l, jnp.int32)])
            sig = 1.0 / (1.0 + jnp.exp(-tot))
            out_v[pl.ds(c * CH + g * LANES, LANES)] = sig

    def work(base, epw):
        nchunk = epw // CH
        pltpu.sync_copy(src_hbm.at[pl.ds(base, epw)], idx_s_all.at[pl.ds(0, epw)])
        pltpu.sync_copy(dst_hbm.at[pl.ds(base, epw)], idx_d_all.at[pl.ds(0, epw)])
        issue(0, rows_u0, rows_v0, sem_u0, sem_v0)

        def pair_body(t, carry):
            c0 = 2 * t
            c1 = 2 * t + 1
            issue(c1, rows_u1, rows_v1, sem_u1, sem_v1)
            drain(rows_u0, rows_v0, sem_u0, sem_v0)
            compute(c0, rows_u0, rows_v0)

            @pl.when(c1 + 1 < nchunk)
            def _():
                issue(c1 + 1, rows_u0, rows_v0, sem_u0, sem_v0)

            drain(rows_u1, rows_v1, sem_u1, sem_v1)
            compute(c1, rows_u1, rows_v1)
            return carry

        lax.fori_loop(0, nchunk // 2, pair_body, 0)
        pltpu.sync_copy(out_v.at[pl.ds(0, epw)], out_hbm.at[pl.ds(base, epw)])

    @pl.when(ci == 0)
    def _():
        work(si * EPW0, EPW0)

    @pl.when(ci == 1)
    def _():
        work(16 * EPW0 + si * EPW1, EPW1)


_sc_edge_scores = functools.partial(
    pl.kernel,
    mesh=plsc.VectorSubcoreMesh(core_axis_name="c", subcore_axis_name="s"),
    compiler_params=pltpu.CompilerParams(needs_layout_passes=False),
    out_type=jax.ShapeDtypeStruct((E_PAD,), jnp.float32),
    scratch_types=[
        pltpu.VMEM((EPWMAX,), jnp.int32),
        pltpu.VMEM((EPWMAX,), jnp.int32),
        pltpu.VMEM((CH, D2 // 2), jnp.int32),
        pltpu.VMEM((CH, D2 // 2), jnp.int32),
        pltpu.VMEM((CH, D2 // 2), jnp.int32),
        pltpu.VMEM((CH, D2 // 2), jnp.int32),
        pltpu.VMEM((LANES, LANES), jnp.float32),
        pltpu.VMEM((EPWMAX,), jnp.float32),
        pltpu.SemaphoreType.DMA,
        pltpu.SemaphoreType.DMA,
        pltpu.SemaphoreType.DMA,
        pltpu.SemaphoreType.DMA,
    ],
)(_sc_body)


def kernel(z_in, z_out, z_self, edge_index, W_in, b_in, W_out, b_out):
    U, V = _build_uv(z_self, z_out, z_in,
                     W_in.T, W_out.T,
                     b_in.reshape(1, D), b_out.reshape(1, D))
    ei = edge_index.astype(jnp.int32)
    pad = jnp.zeros((E_PAD - E,), jnp.int32)
    src = jnp.concatenate([ei[0], pad])
    dst = jnp.concatenate([ei[1], pad])
    out = _sc_edge_scores(U, V, src, dst)
    return out[:E]


# speedup vs baseline: 1.2539x; 1.0146x over previous
import functools

import jax
import jax.numpy as jnp
from jax import lax
from jax.experimental import pallas as pl
from jax.experimental.pallas import tpu as pltpu
from jax.experimental.pallas import tpu_sc as plsc

N = 10000
E = 160000
D = 256
D2 = 2 * D

NW = 32
EPW0 = 6272
EPW1 = 3968
E_PAD = 16 * (EPW0 + EPW1)
CH = 32
EPWMAX = max(EPW0, EPW1)
LANES = 16
_MASK_HI = jnp.int32(-65536)


def _pack_pair(lo_f32, hi_f32):
    lo = lax.bitcast_convert_type(
        lo_f32.astype(jnp.bfloat16), jnp.uint16).astype(jnp.uint32)
    hi = lax.bitcast_convert_type(
        hi_f32.astype(jnp.bfloat16), jnp.uint16).astype(jnp.uint32)
    return lax.bitcast_convert_type(lo | (hi << jnp.uint32(16)), jnp.int32)


def _uv_body(zs, zo, zi, wit, wot, bi, bo, u, v):
    a_in = jnp.dot(zs[...], wit[...], preferred_element_type=jnp.float32) + bi[...]
    a_out = jnp.dot(zs[...], wot[...], preferred_element_type=jnp.float32) + bo[...]
    u[...] = _pack_pair(zo[...], a_in)
    v[...] = _pack_pair(0.5 * a_out, 0.5 * zi[...])


def _build_uv(z_self, z_out, z_in, W_in_T, W_out_T, b_in, b_out):
    R = 1000
    grid = (N // R,)
    return pl.pallas_call(
        _uv_body,
        grid=grid,
        in_specs=[
            pl.BlockSpec((R, D), lambda i: (i, 0)),
            pl.BlockSpec((R, D), lambda i: (i, 0)),
            pl.BlockSpec((R, D), lambda i: (i, 0)),
            pl.BlockSpec((D, D), lambda i: (0, 0)),
            pl.BlockSpec((D, D), lambda i: (0, 0)),
            pl.BlockSpec((1, D), lambda i: (0, 0)),
            pl.BlockSpec((1, D), lambda i: (0, 0)),
        ],
        out_specs=[
            pl.BlockSpec((R, D), lambda i: (i, 0)),
            pl.BlockSpec((R, D), lambda i: (i, 0)),
        ],
        out_shape=[
            jax.ShapeDtypeStruct((N, D), jnp.int32),
            jax.ShapeDtypeStruct((N, D), jnp.int32),
        ],
    )(z_self, z_out, z_in, W_in_T, W_out_T, b_in, b_out)


def _sc_body(u_hbm, v_hbm, src_hbm, dst_hbm, out_hbm,
             idx_s_all, idx_d_all, rows_u0, rows_v0, rows_u1, rows_v1,
             accs, out_v, sem_u0, sem_v0, sem_u1, sem_v1):
    ci = lax.axis_index("c")
    si = lax.axis_index("s")
    lane_iota = lax.iota(jnp.int32, LANES)

    def issue(c, ru, rv, su, sv):
        pltpu.async_copy(u_hbm.at[idx_s_all.at[pl.ds(c * CH, CH)]], ru, su)
        pltpu.async_copy(v_hbm.at[idx_d_all.at[pl.ds(c * CH, CH)]], rv, sv)

    def drain(ru, rv, su, sv):
        pltpu.make_async_copy(u_hbm.at[pl.ds(0, CH)], ru, su).wait()
        pltpu.make_async_copy(v_hbm.at[pl.ds(0, CH)], rv, sv).wait()

    def compute(c, ru, rv):
        for g in range(CH // LANES):
            def edge_body(e, carry2):
                r = g * LANES + e
                lanes_acc = [None, None, None, None]
                for j in range(D2 // (2 * LANES)):
                    uw = ru[r, pl.ds(j * LANES, LANES)]
                    vw = rv[r, pl.ds(j * LANES, LANES)]
                    ue = plsc.bitcast(lax.shift_left(uw, 16), jnp.float32)
                    uo = plsc.bitcast(jnp.bitwise_and(uw, _MASK_HI), jnp.float32)
                    ve = plsc.bitcast(lax.shift_left(vw, 16), jnp.float32)
                    vo = plsc.bitcast(jnp.bitwise_and(vw, _MASK_HI), jnp.float32)
                    k = 2 * (j % 2)
                    pe = ue * ve
                    po = uo * vo
                    lanes_acc[k] = pe if lanes_acc[k] is None else lanes_acc[k] + pe
                    lanes_acc[k + 1] = po if lanes_acc[k + 1] is None else lanes_acc[k + 1] + po
                acc = ((lanes_acc[0] + lanes_acc[1])
                       + (lanes_acc[2] + lanes_acc[3]))
                accs[e, :] = acc
                return carry2

            lax.fori_loop(0, LANES, edge_body, 0)
            tot = plsc.load_gather(
                accs, [lane_iota, jnp.zeros((LANES,), jnp.int32)])
            for l in range(1, LANES):
                tot = tot + plsc.load_gather(
                    accs, [lane_iota, jnp.full((LANES,), l, jnp.int32)])
            sig = 1.0 / (1.0 + jnp.exp(-tot))
            out_v[pl.ds(c * CH + g * LANES, LANES)] = sig

    def work(base, epw):
        nchunk = epw // CH
        pltpu.sync_copy(src_hbm.at[pl.ds(base, epw)], idx_s_all.at[pl.ds(0, epw)])
        pltpu.sync_copy(dst_hbm.at[pl.ds(base, epw)], idx_d_all.at[pl.ds(0, epw)])
        issue(0, rows_u0, rows_v0, sem_u0, sem_v0)

        def pair_body(t, carry):
            c0 = 2 * t
            c1 = 2 * t + 1
            issue(c1, rows_u1, rows_v1, sem_u1, sem_v1)
            drain(rows_u0, rows_v0, sem_u0, sem_v0)
            compute(c0, rows_u0, rows_v0)

            @pl.when(c1 + 1 < nchunk)
            def _():
                issue(c1 + 1, rows_u0, rows_v0, sem_u0, sem_v0)

            drain(rows_u1, rows_v1, sem_u1, sem_v1)
            compute(c1, rows_u1, rows_v1)
            return carry

        lax.fori_loop(0, nchunk // 2, pair_body, 0)
        pltpu.sync_copy(out_v.at[pl.ds(0, epw)], out_hbm.at[pl.ds(base, epw)])

    @pl.when(ci == 0)
    def _():
        work(si * EPW0, EPW0)

    @pl.when(ci == 1)
    def _():
        work(16 * EPW0 + si * EPW1, EPW1)


_sc_edge_scores = functools.partial(
    pl.kernel,
    mesh=plsc.VectorSubcoreMesh(core_axis_name="c", subcore_axis_name="s"),
    compiler_params=pltpu.CompilerParams(needs_layout_passes=False),
    out_type=jax.ShapeDtypeStruct((E_PAD,), jnp.float32),
    scratch_types=[
        pltpu.VMEM((EPWMAX,), jnp.int32),
        pltpu.VMEM((EPWMAX,), jnp.int32),
        pltpu.VMEM((CH, D2 // 2), jnp.int32),
        pltpu.VMEM((CH, D2 // 2), jnp.int32),
        pltpu.VMEM((CH, D2 // 2), jnp.int32),
        pltpu.VMEM((CH, D2 // 2), jnp.int32),
        pltpu.VMEM((LANES, LANES), jnp.float32),
        pltpu.VMEM((EPWMAX,), jnp.float32),
        pltpu.SemaphoreType.DMA,
        pltpu.SemaphoreType.DMA,
        pltpu.SemaphoreType.DMA,
        pltpu.SemaphoreType.DMA,
    ],
)(_sc_body)


def kernel(z_in, z_out, z_self, edge_index, W_in, b_in, W_out, b_out):
    U, V = _build_uv(z_self, z_out, z_in,
                     W_in.T, W_out.T,
                     b_in.reshape(1, D), b_out.reshape(1, D))
    ei = edge_index.astype(jnp.int32)
    pad = jnp.zeros((E_PAD - E,), jnp.int32)
    src = jnp.concatenate([ei[0], pad])
    dst = jnp.concatenate([ei[1], pad])
    out = _sc_edge_scores(U, V, src, dst)
    return out[:E]
